# Initial kernel scaffold; baseline (speedup 1.0000x reference)
import functools

import jax
import jax.numpy as jnp
from jax import lax
from jax.experimental import pallas as pl
from jax.experimental.pallas import tpu as pltpu
from jax.experimental.pallas import tpu_sc as plsc

EPS_LUB = 0.0001

N_NODES = 100000
N_EDGES = 6400000
NUM_CORES = 2
NUM_SUBCORES = 16
CHUNK = 4000
EDGES_PER_TILE = N_EDGES // NUM_SUBCORES
NUM_CHUNKS = EDGES_PER_TILE // CHUNK
VROWS = 4000
NUM_VTILES = N_NODES // VROWS


def _body(f_v_hbm, fcols_hbm, eflat_hbm, out_hbm, tbl, ebuf, obuf, vbuf):
    c = lax.axis_index("c")
    s = lax.axis_index("s")
    wid = c * NUM_SUBCORES + s

    pltpu.sync_copy(fcols_hbm.at[c], tbl)

    @pl.when(wid < NUM_VTILES)
    def _copy_prefix():
        pltpu.sync_copy(f_v_hbm.at[pl.ds(wid * VROWS, VROWS), :], vbuf)
        pltpu.sync_copy(vbuf, out_hbm.at[pl.ds(wid * VROWS, VROWS), :])

    lanes = lax.iota(jnp.int32, 16)
    swap = lanes ^ 1
    pat = lanes >> 1
    zeros = lanes & 0

    tile_base = s * EDGES_PER_TILE

    def chunk_body(k, carry):
        ebase = tile_base + k * CHUNK
        pltpu.sync_copy(eflat_hbm.at[pl.ds(ebase * 2, 2 * CHUNK)], ebuf)

        def inner(j, carry2):
            va = ebuf[pl.ds(j * 32, 16)]
            vb = ebuf[pl.ds(j * 32 + 16, 16)]
            ga = plsc.load_gather(tbl, [va])
            gb = plsc.load_gather(tbl, [vb])
            sa = jnp.take(ga, swap, mode="promise_in_bounds")
            sb = jnp.take(gb, swap, mode="promise_in_bounds")
            ma = jnp.maximum(ga, sa) + EPS_LUB
            mb = jnp.maximum(gb, sb) + EPS_LUB
            base_o = j * 16
            plsc.store_scatter(obuf, [base_o + pat, zeros], ma)
            plsc.store_scatter(obuf, [base_o + 8 + pat, zeros], mb)
            return carry2

        lax.fori_loop(0, CHUNK // 16, inner, 0, unroll=4)
        pltpu.sync_copy(
            obuf, out_hbm.at[pl.ds(N_NODES + ebase, CHUNK), pl.ds(c, 1)]
        )
        return carry

    lax.fori_loop(0, NUM_CHUNKS, chunk_body, 0)


@jax.jit
def _run(f_v, fcols, eflat):
    mesh = plsc.VectorSubcoreMesh(
        core_axis_name="c", subcore_axis_name="s"
    )
    k = functools.partial(
        pl.kernel,
        mesh=mesh,
        out_type=jax.ShapeDtypeStruct((N_NODES + N_EDGES, 2), jnp.float32),
        scratch_types=[
            pltpu.VMEM((N_NODES,), jnp.float32),
            pltpu.VMEM((2 * CHUNK,), jnp.int32),
            pltpu.VMEM((CHUNK, 1), jnp.float32),
            pltpu.VMEM((VROWS, 2), jnp.float32),
        ],
    )(_body)
    return k(f_v, fcols, eflat)


def kernel(f_v, edges):
    f_v = f_v.astype(jnp.float32)
    fcols = f_v.T
    eflat = edges.astype(jnp.int32).reshape(-1)
    return _run(f_v, fcols, eflat)


# speedup vs baseline: 25.9656x; 25.9656x over previous
import functools

import jax
import jax.numpy as jnp
from jax import lax
from jax.experimental import pallas as pl
from jax.experimental.pallas import tpu as pltpu
from jax.experimental.pallas import tpu_sc as plsc

EPS_LUB = 0.0001

_GATHER_DNUMS = lax.GatherDimensionNumbers(
    offset_dims=(), collapsed_slice_dims=(0,), start_index_map=(0,)
)


def _permute16(v, idx):
    return lax.gather(
        v,
        idx[:, None],
        _GATHER_DNUMS,
        slice_sizes=(1,),
        mode=lax.GatherScatterMode.PROMISE_IN_BOUNDS,
    )


N_NODES = 100000
N_EDGES = 6400000
NUM_CORES = 2
NUM_SUBCORES = 16
CHUNK = 4000
EDGES_PER_TILE = N_EDGES // NUM_SUBCORES
NUM_CHUNKS = EDGES_PER_TILE // CHUNK
VCOPY = N_NODES // 4


def _body(fcols_hbm, eflat_hbm, out_hbm, tbl, ebuf, obuf):
    c = lax.axis_index("c")
    s = lax.axis_index("s")

    pltpu.sync_copy(fcols_hbm.at[c], tbl)

    @pl.when(s < 4)
    def _copy_prefix():
        pltpu.sync_copy(
            tbl.at[pl.ds(s * VCOPY, VCOPY)],
            out_hbm.at[c, pl.ds(s * VCOPY, VCOPY)],
        )

    lanes = lax.iota(jnp.int32, 16)
    swap = lanes ^ 1
    pat = lanes >> 1

    tile_base = s * EDGES_PER_TILE

    def chunk_body(k, carry):
        ebase = tile_base + k * CHUNK
        pltpu.sync_copy(eflat_hbm.at[pl.ds(ebase * 2, 2 * CHUNK)], ebuf)

        def inner(j, carry2):
            va = ebuf[pl.ds(j * 32, 16)]
            vb = ebuf[pl.ds(j * 32 + 16, 16)]
            ga = plsc.load_gather(tbl, [va])
            gb = plsc.load_gather(tbl, [vb])
            sa = _permute16(ga, swap)
            sb = _permute16(gb, swap)
            ma = jnp.maximum(ga, sa) + EPS_LUB
            mb = jnp.maximum(gb, sb) + EPS_LUB
            base_o = j * 16
            plsc.store_scatter(obuf, [base_o + pat], ma)
            plsc.store_scatter(obuf, [base_o + 8 + pat], mb)
            return carry2

        lax.fori_loop(0, CHUNK // 16, inner, 0, unroll=4)
        pltpu.sync_copy(obuf, out_hbm.at[c, pl.ds(N_NODES + ebase, CHUNK)])
        return carry

    lax.fori_loop(0, NUM_CHUNKS, chunk_body, 0)


@jax.jit
def _run(fcols, eflat):
    mesh = plsc.VectorSubcoreMesh(
        core_axis_name="c", subcore_axis_name="s"
    )
    k = functools.partial(
        pl.kernel,
        mesh=mesh,
        out_type=jax.ShapeDtypeStruct((2, N_NODES + N_EDGES), jnp.float32),
        scratch_types=[
            pltpu.VMEM((N_NODES,), jnp.float32),
            pltpu.VMEM((2 * CHUNK,), jnp.int32),
            pltpu.VMEM((CHUNK,), jnp.float32),
        ],
        compiler_params=pltpu.CompilerParams(
            use_tc_tiling_on_sc=False, needs_layout_passes=False
        ),
    )(_body)
    return k(fcols, eflat)


def kernel(f_v, edges):
    f_v = f_v.astype(jnp.float32)
    fcols = f_v.T
    eflat = edges.astype(jnp.int32).reshape(-1)
    out_t = _run(fcols, eflat)
    return out_t.T
